# Initial kernel scaffold; baseline (speedup 1.0000x reference)
import jax
import jax.numpy as jnp
from jax import lax
from jax.experimental import pallas as pl
from jax.experimental.pallas import tpu as pltpu
from jax.experimental.pallas import tpu_sc as plsc

NN = 10000
EDGES = 320000
DIN = 128
HID = 32
NC, NS = 2, 16
NW = NC * NS
CHUNK = 128
CPW = 79
EPAD = NW * CPW * CHUNK
NPAD = 10016
RPW = NPAD // NS

_f32 = jnp.float32


def _edge_pass(with_counts):
    out_types = [jax.ShapeDtypeStruct((NC, NPAD, HID), _f32)]
    scratch = [
        pltpu.VMEM((CPW, CHUNK), jnp.int32),
        pltpu.VMEM((CPW, CHUNK), jnp.int32),
        pltpu.VMEM((CHUNK, HID), _f32),
        pltpu.VMEM((RPW, HID), _f32),
    ]
    if with_counts:
        out_types.append(jax.ShapeDtypeStruct((NC, NPAD, 16), _f32))
        scratch.append(pltpu.VMEM((CHUNK, 16), _f32))
        scratch.append(pltpu.VMEM((RPW, 16), _f32))
    scratch.append(pltpu.VMEM_SHARED((NPAD, HID), _f32))
    if with_counts:
        scratch.append(pltpu.VMEM_SHARED((NPAD, 16), _f32))

    def body(*refs):
        if with_counts:
            (table, srci, dsti, out, cnt_out,
             src_v, dst_v, rows_v, zbuf, ones_v, zbuf16, acc, cntacc) = refs
        else:
            (table, srci, dsti, out,
             src_v, dst_v, rows_v, zbuf, acc) = refs

        cid = lax.axis_index("c")
        sid = lax.axis_index("s")
        wid = cid * NS + sid

        @pl.loop(0, RPW)
        def _(i):
            zbuf.at[i, pl.ds(0, 16)][...] = jnp.zeros((16,), _f32)
            zbuf.at[i, pl.ds(16, 16)][...] = jnp.zeros((16,), _f32)

        pltpu.sync_copy(zbuf, acc.at[pl.ds(sid * RPW, RPW)])
        if with_counts:
            @pl.loop(0, RPW)
            def _(i):
                zbuf16.at[i][...] = jnp.zeros((16,), _f32)

            pltpu.sync_copy(zbuf16, cntacc.at[pl.ds(sid * RPW, RPW)])

            @pl.loop(0, CHUNK)
            def _(i):
                ones_v.at[i][...] = jnp.ones((16,), _f32)

        pltpu.sync_copy(srci.at[wid], src_v)
        pltpu.sync_copy(dsti.at[wid], dst_v)

        plsc.subcore_barrier()

        @pl.loop(0, CPW)
        def _(j):
            pltpu.sync_copy(table.at[src_v.at[j]], rows_v)
            pltpu.sync_copy(rows_v, acc.at[dst_v.at[j]], add=True)
            if with_counts:
                pltpu.sync_copy(ones_v, cntacc.at[dst_v.at[j]], add=True)

        plsc.subcore_barrier()

        pltpu.sync_copy(acc.at[pl.ds(sid * RPW, RPW)],
                        out.at[cid, pl.ds(sid * RPW, RPW)])
        if with_counts:
            pltpu.sync_copy(cntacc.at[pl.ds(sid * RPW, RPW)],
                            cnt_out.at[cid, pl.ds(sid * RPW, RPW)])

    mesh = plsc.VectorSubcoreMesh(
        core_axis_name="c", subcore_axis_name="s", num_cores=NC, num_subcores=NS
    )
    return pl.kernel(
        body,
        out_type=tuple(out_types) if with_counts else out_types[0],
        mesh=mesh,
        scratch_types=scratch,
    )


_edge_pass1 = _edge_pass(with_counts=True)
_edge_pass2 = _edge_pass(with_counts=False)


def _dense_in_body(x_ref, wl_ref, wr_ref, b_ref, p_ref, r_ref):
    xv = x_ref[...]
    p_ref[...] = lax.dot(xv, wl_ref[...], precision=lax.Precision.HIGHEST,
                         preferred_element_type=_f32)
    r_ref[...] = lax.dot(xv, wr_ref[...], precision=lax.Precision.HIGHEST,
                         preferred_element_type=_f32) + b_ref[...]


_dense_in = pl.pallas_call(
    _dense_in_body,
    out_shape=(jax.ShapeDtypeStruct((NN, HID), _f32),
               jax.ShapeDtypeStruct((NN, HID), _f32)),
)


def _mid_body(s_ref, c_ref, r_ref, wl_ref, wr_ref, b_ref, p_ref, r2_ref):
    s = s_ref[0, :NN, :] + s_ref[1, :NN, :]
    cnt = c_ref[0, :NN, 0:1] + c_ref[1, :NN, 0:1]
    mean = s / jnp.maximum(cnt, 1.0)
    h = jnp.maximum(mean + r_ref[...], 0.0)
    p_ref[...] = lax.dot(h, wl_ref[...], precision=lax.Precision.HIGHEST,
                         preferred_element_type=_f32)
    r2_ref[...] = lax.dot(h, wr_ref[...], precision=lax.Precision.HIGHEST,
                          preferred_element_type=_f32) + b_ref[...]


_mid = pl.pallas_call(
    _mid_body,
    out_shape=(jax.ShapeDtypeStruct((NN, HID), _f32),
               jax.ShapeDtypeStruct((NN, HID), _f32)),
)


def _head_body(s_ref, c_ref, r_ref, w_ref, b_ref, o_ref):
    s = s_ref[0, :NN, :] + s_ref[1, :NN, :]
    cnt = c_ref[0, :NN, 0:1] + c_ref[1, :NN, 0:1]
    mean = s / jnp.maximum(cnt, 1.0)
    h = jnp.maximum(mean + r_ref[...], 0.0)
    o_ref[...] = lax.dot(h, w_ref[...], precision=lax.Precision.HIGHEST,
                         preferred_element_type=_f32) + b_ref[...]


_head = pl.pallas_call(
    _head_body,
    out_shape=jax.ShapeDtypeStruct((NN, 11), _f32),
)


def kernel(x, edge_index, W1l, b1, W1r, W2l, b2, W2r, Wh, bh, Wm, bm):
    src = edge_index[0]
    dst = edge_index[1]
    npad = EPAD - EDGES
    src_r = jnp.concatenate(
        [src, jnp.zeros((npad,), jnp.int32)]).reshape(NW, CPW, CHUNK)
    dst_r = jnp.concatenate(
        [dst, jnp.full((npad,), NN, jnp.int32)]).reshape(NW, CPW, CHUNK)

    p1, r1 = _dense_in(x, W1l, W1r, b1.reshape(1, HID))
    s1, cnt = _edge_pass1(p1, src_r, dst_r)
    p2, r2 = _mid(s1, cnt, r1, W2l, W2r, b2.reshape(1, HID))
    s2 = _edge_pass2(p2, src_r, dst_r)
    whm = jnp.concatenate([Wh, Wm], axis=1)
    bhm = jnp.concatenate([bh, bm]).reshape(1, 11)
    o = _head(s2, cnt, r2, whm, bhm)
    return (o[:, :3], o[:, 3:11])


# speedup vs baseline: 9.8505x; 9.8505x over previous
import jax
import jax.numpy as jnp
from jax import lax
from jax.experimental import pallas as pl
from jax.experimental.pallas import tpu as pltpu
from jax.experimental.pallas import tpu_sc as plsc

NN = 10000
EDGES = 320000
DIN = 128
HID = 32
NC, NS = 2, 16
NW = NC * NS
CHUNK = 128
CPW = 79
EPAD = NW * CPW * CHUNK
NPAD = 10240
RPW = NPAD // NS

_f32 = jnp.float32


def _edge_pass(with_counts):
    out_types = [jax.ShapeDtypeStruct((NC, NPAD, HID), _f32)]
    scratch = [
        pltpu.VMEM((CPW, CHUNK), jnp.int32),
        pltpu.VMEM((CPW, CHUNK), jnp.int32),
        pltpu.VMEM((CHUNK, HID), _f32),
        pltpu.VMEM((RPW, HID), _f32),
    ]
    if with_counts:
        out_types.append(jax.ShapeDtypeStruct((NC, NPAD, 16), _f32))
        scratch.append(pltpu.VMEM((CHUNK, 16), _f32))
        scratch.append(pltpu.VMEM((RPW, 16), _f32))
    scratch.append(pltpu.VMEM_SHARED((NPAD, HID), _f32))
    if with_counts:
        scratch.append(pltpu.VMEM_SHARED((NPAD, 16), _f32))

    def body(*refs):
        if with_counts:
            (table, srci, dsti, out, cnt_out,
             src_v, dst_v, rows_v, zbuf, ones_v, zbuf16, acc, cntacc) = refs
        else:
            (table, srci, dsti, out,
             src_v, dst_v, rows_v, zbuf, acc) = refs

        cid = lax.axis_index("c")
        sid = lax.axis_index("s")
        wid = cid * NS + sid

        @pl.loop(0, RPW)
        def _(i):
            zbuf.at[i, pl.ds(0, 16)][...] = jnp.zeros((16,), _f32)
            zbuf.at[i, pl.ds(16, 16)][...] = jnp.zeros((16,), _f32)

        pltpu.sync_copy(zbuf, acc.at[pl.ds(sid * RPW, RPW)])
        if with_counts:
            @pl.loop(0, RPW)
            def _(i):
                zbuf16.at[i][...] = jnp.zeros((16,), _f32)

            pltpu.sync_copy(zbuf16, cntacc.at[pl.ds(sid * RPW, RPW)])

            @pl.loop(0, CHUNK)
            def _(i):
                ones_v.at[i][...] = jnp.ones((16,), _f32)

        pltpu.sync_copy(srci.at[wid], src_v)
        pltpu.sync_copy(dsti.at[wid], dst_v)

        plsc.subcore_barrier()

        @pl.loop(0, CPW)
        def _(j):
            pltpu.sync_copy(table.at[src_v.at[j]], rows_v)
            pltpu.sync_copy(rows_v, acc.at[dst_v.at[j]], add=True)
            if with_counts:
                pltpu.sync_copy(ones_v, cntacc.at[dst_v.at[j]], add=True)

        plsc.subcore_barrier()

        pltpu.sync_copy(acc.at[pl.ds(sid * RPW, RPW)],
                        out.at[cid, pl.ds(sid * RPW, RPW)])
        if with_counts:
            pltpu.sync_copy(cntacc.at[pl.ds(sid * RPW, RPW)],
                            cnt_out.at[cid, pl.ds(sid * RPW, RPW)])

    mesh = plsc.VectorSubcoreMesh(
        core_axis_name="c", subcore_axis_name="s", num_cores=NC, num_subcores=NS
    )
    return pl.kernel(
        body,
        out_type=tuple(out_types) if with_counts else out_types[0],
        mesh=mesh,
        scratch_types=scratch,
        compiler_params=pltpu.CompilerParams(use_tc_tiling_on_sc=False),
    )


_edge_pass1 = _edge_pass(with_counts=True)
_edge_pass2 = _edge_pass(with_counts=False)


def _dense_in_body(x_ref, wl_ref, wr_ref, b_ref, p_ref, r_ref):
    xv = x_ref[...]
    p_ref[...] = lax.dot(xv, wl_ref[...], precision=lax.Precision.HIGHEST,
                         preferred_element_type=_f32)
    r_ref[...] = lax.dot(xv, wr_ref[...], precision=lax.Precision.HIGHEST,
                         preferred_element_type=_f32) + b_ref[...]


_dense_in = pl.pallas_call(
    _dense_in_body,
    out_shape=(jax.ShapeDtypeStruct((NN, HID), _f32),
               jax.ShapeDtypeStruct((NN, HID), _f32)),
)


def _mid_body(s_ref, c_ref, r_ref, wl_ref, wr_ref, b_ref, p_ref, r2_ref):
    s = s_ref[0, :NN, :] + s_ref[1, :NN, :]
    cnt = c_ref[0, :NN, 0:1] + c_ref[1, :NN, 0:1]
    mean = s / jnp.maximum(cnt, 1.0)
    h = jnp.maximum(mean + r_ref[...], 0.0)
    p_ref[...] = lax.dot(h, wl_ref[...], precision=lax.Precision.HIGHEST,
                         preferred_element_type=_f32)
    r2_ref[...] = lax.dot(h, wr_ref[...], precision=lax.Precision.HIGHEST,
                          preferred_element_type=_f32) + b_ref[...]


_mid = pl.pallas_call(
    _mid_body,
    out_shape=(jax.ShapeDtypeStruct((NN, HID), _f32),
               jax.ShapeDtypeStruct((NN, HID), _f32)),
)


def _head_body(s_ref, c_ref, r_ref, w_ref, b_ref, o_ref):
    s = s_ref[0, :NN, :] + s_ref[1, :NN, :]
    cnt = c_ref[0, :NN, 0:1] + c_ref[1, :NN, 0:1]
    mean = s / jnp.maximum(cnt, 1.0)
    h = jnp.maximum(mean + r_ref[...], 0.0)
    o_ref[...] = lax.dot(h, w_ref[...], precision=lax.Precision.HIGHEST,
                         preferred_element_type=_f32) + b_ref[...]


_head = pl.pallas_call(
    _head_body,
    out_shape=jax.ShapeDtypeStruct((NN, 11), _f32),
)


def kernel(x, edge_index, W1l, b1, W1r, W2l, b2, W2r, Wh, bh, Wm, bm):
    src = edge_index[0]
    dst = edge_index[1]
    npad = EPAD - EDGES
    src_r = jnp.concatenate(
        [src, jnp.zeros((npad,), jnp.int32)]).reshape(NW, CPW, CHUNK)
    dst_r = jnp.concatenate(
        [dst, jnp.full((npad,), NN, jnp.int32)]).reshape(NW, CPW, CHUNK)

    p1, r1 = _dense_in(x, W1l, W1r, b1.reshape(1, HID))
    s1, cnt = _edge_pass1(p1, src_r, dst_r)
    p2, r2 = _mid(s1, cnt, r1, W2l, W2r, b2.reshape(1, HID))
    s2 = _edge_pass2(p2, src_r, dst_r)
    whm = jnp.concatenate([Wh, Wm], axis=1)
    bhm = jnp.concatenate([bh, bm]).reshape(1, 11)
    o = _head(s2, cnt, r2, whm, bhm)
    return (o[:, :3], o[:, 3:11])


# speedup vs baseline: 10.4663x; 1.0625x over previous
import jax
import jax.numpy as jnp
from jax import lax
from jax.experimental import pallas as pl
from jax.experimental.pallas import tpu as pltpu
from jax.experimental.pallas import tpu_sc as plsc

NN = 10000
EDGES = 320000
DIN = 128
HID = 32
NC, NS = 2, 16
NW = NC * NS
CHUNK = 128
CPW = 80
NBUF = 4
EPAD = NW * CPW * CHUNK
NPAD = 10240
RPW = NPAD // NS

_f32 = jnp.float32


def _edge_pass(with_counts):
    out_types = [jax.ShapeDtypeStruct((NC, NPAD, HID), _f32)]
    scratch = [
        pltpu.VMEM((CPW, CHUNK), jnp.int32),
        pltpu.VMEM((CPW, CHUNK), jnp.int32),
    ]
    scratch += [pltpu.VMEM((CHUNK, HID), _f32) for _ in range(NBUF)]
    scratch.append(pltpu.VMEM((RPW, HID), _f32))
    scratch += [pltpu.SemaphoreType.DMA] * NBUF
    if with_counts:
        out_types.append(jax.ShapeDtypeStruct((NC, NPAD, 16), _f32))
        scratch.append(pltpu.VMEM((CHUNK, 16), _f32))
        scratch.append(pltpu.VMEM((RPW, 16), _f32))
        scratch.append(pltpu.SemaphoreType.DMA)
    scratch.append(pltpu.VMEM_SHARED((NPAD, HID), _f32))
    if with_counts:
        scratch.append(pltpu.VMEM_SHARED((NPAD, 16), _f32))

    def body(*refs):
        if with_counts:
            (table, srci, dsti, out, cnt_out,
             src_v, dst_v, *mid, ones_v, zbuf16, csem,
             acc, cntacc) = refs
        else:
            (table, srci, dsti, out,
             src_v, dst_v, *mid, acc) = refs
        rows = mid[:NBUF]
        zbuf = mid[NBUF]
        gsem = mid[NBUF + 1:]

        cid = lax.axis_index("c")
        sid = lax.axis_index("s")
        wid = cid * NS + sid

        @pl.loop(0, RPW)
        def _(i):
            zbuf.at[i, pl.ds(0, 16)][...] = jnp.zeros((16,), _f32)
            zbuf.at[i, pl.ds(16, 16)][...] = jnp.zeros((16,), _f32)

        pltpu.sync_copy(zbuf, acc.at[pl.ds(sid * RPW, RPW)])
        if with_counts:
            @pl.loop(0, RPW)
            def _(i):
                zbuf16.at[i][...] = jnp.zeros((16,), _f32)

            pltpu.sync_copy(zbuf16, cntacc.at[pl.ds(sid * RPW, RPW)])

            @pl.loop(0, CHUNK)
            def _(i):
                ones_v.at[i][...] = jnp.ones((16,), _f32)

        pltpu.sync_copy(srci.at[wid], src_v)
        pltpu.sync_copy(dsti.at[wid], dst_v)

        plsc.subcore_barrier()

        for b in range(NBUF):
            pltpu.async_copy(table.at[src_v.at[b]], rows[b], gsem[b])

        @pl.loop(0, CPW, step=NBUF)
        def _(j):
            for b in range(NBUF):
                c = j + b
                pltpu.make_async_copy(
                    table.at[src_v.at[c]], rows[b], gsem[b]).wait()
                pltpu.sync_copy(rows[b], acc.at[dst_v.at[c]], add=True)

                @pl.when(c + NBUF < CPW)
                def _():
                    pltpu.async_copy(
                        table.at[src_v.at[c + NBUF]], rows[b], gsem[b])

                if with_counts:
                    pltpu.sync_copy(ones_v, cntacc.at[dst_v.at[c]], add=True)

        plsc.subcore_barrier()

        pltpu.sync_copy(acc.at[pl.ds(sid * RPW, RPW)],
                        out.at[cid, pl.ds(sid * RPW, RPW)])
        if with_counts:
            pltpu.sync_copy(cntacc.at[pl.ds(sid * RPW, RPW)],
                            cnt_out.at[cid, pl.ds(sid * RPW, RPW)])

    mesh = plsc.VectorSubcoreMesh(
        core_axis_name="c", subcore_axis_name="s", num_cores=NC, num_subcores=NS
    )
    return pl.kernel(
        body,
        out_type=tuple(out_types) if with_counts else out_types[0],
        mesh=mesh,
        scratch_types=scratch,
        compiler_params=pltpu.CompilerParams(use_tc_tiling_on_sc=False),
    )


_edge_pass1 = _edge_pass(with_counts=True)
_edge_pass2 = _edge_pass(with_counts=False)


def _dense_in_body(x_ref, wl_ref, wr_ref, b_ref, p_ref, r_ref):
    xv = x_ref[...]
    p_ref[...] = lax.dot(xv, wl_ref[...], precision=lax.Precision.HIGHEST,
                         preferred_element_type=_f32)
    r_ref[...] = lax.dot(xv, wr_ref[...], precision=lax.Precision.HIGHEST,
                         preferred_element_type=_f32) + b_ref[...]


_dense_in = pl.pallas_call(
    _dense_in_body,
    out_shape=(jax.ShapeDtypeStruct((NN, HID), _f32),
               jax.ShapeDtypeStruct((NN, HID), _f32)),
)


def _mid_body(s_ref, c_ref, r_ref, wl_ref, wr_ref, b_ref, p_ref, r2_ref):
    s = s_ref[0, :NN, :] + s_ref[1, :NN, :]
    cnt = c_ref[0, :NN, 0:1] + c_ref[1, :NN, 0:1]
    mean = s / jnp.maximum(cnt, 1.0)
    h = jnp.maximum(mean + r_ref[...], 0.0)
    p_ref[...] = lax.dot(h, wl_ref[...], precision=lax.Precision.HIGHEST,
                         preferred_element_type=_f32)
    r2_ref[...] = lax.dot(h, wr_ref[...], precision=lax.Precision.HIGHEST,
                          preferred_element_type=_f32) + b_ref[...]


_mid = pl.pallas_call(
    _mid_body,
    out_shape=(jax.ShapeDtypeStruct((NN, HID), _f32),
               jax.ShapeDtypeStruct((NN, HID), _f32)),
)


def _head_body(s_ref, c_ref, r_ref, w_ref, b_ref, o_ref):
    s = s_ref[0, :NN, :] + s_ref[1, :NN, :]
    cnt = c_ref[0, :NN, 0:1] + c_ref[1, :NN, 0:1]
    mean = s / jnp.maximum(cnt, 1.0)
    h = jnp.maximum(mean + r_ref[...], 0.0)
    o_ref[...] = lax.dot(h, w_ref[...], precision=lax.Precision.HIGHEST,
                         preferred_element_type=_f32) + b_ref[...]


_head = pl.pallas_call(
    _head_body,
    out_shape=jax.ShapeDtypeStruct((NN, 11), _f32),
)


def kernel(x, edge_index, W1l, b1, W1r, W2l, b2, W2r, Wh, bh, Wm, bm):
    src = edge_index[0]
    dst = edge_index[1]
    npad = EPAD - EDGES
    src_r = jnp.concatenate(
        [src, jnp.zeros((npad,), jnp.int32)]).reshape(NW, CPW, CHUNK)
    dst_r = jnp.concatenate(
        [dst, jnp.full((npad,), NN, jnp.int32)]).reshape(NW, CPW, CHUNK)

    p1, r1 = _dense_in(x, W1l, W1r, b1.reshape(1, HID))
    s1, cnt = _edge_pass1(p1, src_r, dst_r)
    p2, r2 = _mid(s1, cnt, r1, W2l, W2r, b2.reshape(1, HID))
    s2 = _edge_pass2(p2, src_r, dst_r)
    whm = jnp.concatenate([Wh, Wm], axis=1)
    bhm = jnp.concatenate([bh, bm]).reshape(1, 11)
    o = _head(s2, cnt, r2, whm, bhm)
    return (o[:, :3], o[:, 3:11])


# speedup vs baseline: 18.3106x; 1.7495x over previous
import jax
import jax.numpy as jnp
from jax import lax
from jax.experimental import pallas as pl
from jax.experimental.pallas import tpu as pltpu
from jax.experimental.pallas import tpu_sc as plsc

NN = 10000
EDGES = 320000
DIN = 128
HID = 32
NC, NS = 2, 16
NW = NC * NS
CHUNK = 128
CPW = 80
NBUF = 4
EPAD = NW * CPW * CHUNK
NPAD = 10240
RPW = NPAD // NS

_f32 = jnp.float32


def _edge_pass(with_counts):
    out_types = [jax.ShapeDtypeStruct((NC, NPAD, HID), _f32)]
    scratch = [
        pltpu.VMEM((CPW, CHUNK), jnp.int32),
        pltpu.VMEM((CPW, CHUNK), jnp.int32),
    ]
    scratch += [pltpu.VMEM((CHUNK, HID), _f32) for _ in range(NBUF)]
    scratch.append(pltpu.VMEM((RPW, HID), _f32))
    scratch += [pltpu.SemaphoreType.DMA] * NBUF
    if with_counts:
        out_types.append(jax.ShapeDtypeStruct((NC, NPAD, 16), _f32))
        scratch.append(pltpu.VMEM((CHUNK, 16), _f32))
        scratch.append(pltpu.VMEM((RPW, 16), _f32))
        scratch.append(pltpu.SemaphoreType.DMA)
    scratch.append(pltpu.VMEM_SHARED((NPAD, HID), _f32))
    if with_counts:
        scratch.append(pltpu.VMEM_SHARED((NPAD, 16), _f32))

    def body(*refs):
        if with_counts:
            (table, srci, dsti, out, cnt_out,
             src_v, dst_v, *mid, ones_v, zbuf16, csem,
             acc, cntacc) = refs
        else:
            (table, srci, dsti, out,
             src_v, dst_v, *mid, acc) = refs
        rows = mid[:NBUF]
        zbuf = mid[NBUF]
        gsem = mid[NBUF + 1:]

        cid = lax.axis_index("c")
        sid = lax.axis_index("s")
        wid = cid * NS + sid

        @pl.loop(0, RPW)
        def _(i):
            zbuf.at[i, pl.ds(0, 16)][...] = jnp.zeros((16,), _f32)
            zbuf.at[i, pl.ds(16, 16)][...] = jnp.zeros((16,), _f32)

        pltpu.sync_copy(zbuf, acc.at[pl.ds(sid * RPW, RPW)])
        if with_counts:
            @pl.loop(0, RPW)
            def _(i):
                zbuf16.at[i][...] = jnp.zeros((16,), _f32)

            pltpu.sync_copy(zbuf16, cntacc.at[pl.ds(sid * RPW, RPW)])

            @pl.loop(0, CHUNK)
            def _(i):
                ones_v.at[i][...] = jnp.ones((16,), _f32)

        pltpu.sync_copy(srci.at[wid], src_v)
        pltpu.sync_copy(dsti.at[wid], dst_v)

        plsc.subcore_barrier()

        for b in range(NBUF):
            pltpu.async_copy(table.at[src_v.at[b]], rows[b], gsem[b])

        @pl.loop(0, CPW, step=NBUF)
        def _(j):
            for b in range(NBUF):
                c = j + b
                pltpu.make_async_copy(
                    table.at[src_v.at[c]], rows[b], gsem[b]).wait()
                pltpu.sync_copy(rows[b], acc.at[dst_v.at[c]], add=True)

                @pl.when(c + NBUF < CPW)
                def _():
                    pltpu.async_copy(
                        table.at[src_v.at[c + NBUF]], rows[b], gsem[b])

                if with_counts:
                    pltpu.sync_copy(ones_v, cntacc.at[dst_v.at[c]], add=True)

        plsc.subcore_barrier()

        pltpu.sync_copy(acc.at[pl.ds(sid * RPW, RPW)],
                        out.at[cid, pl.ds(sid * RPW, RPW)])
        if with_counts:
            pltpu.sync_copy(cntacc.at[pl.ds(sid * RPW, RPW)],
                            cnt_out.at[cid, pl.ds(sid * RPW, RPW)])

    mesh = plsc.VectorSubcoreMesh(
        core_axis_name="c", subcore_axis_name="s", num_cores=NC, num_subcores=NS
    )
    return pl.kernel(
        body,
        out_type=tuple(out_types) if with_counts else out_types[0],
        mesh=mesh,
        scratch_types=scratch,
        compiler_params=pltpu.CompilerParams(use_tc_tiling_on_sc=False),
    )


_edge_pass1 = _edge_pass(with_counts=True)
_edge_pass2 = _edge_pass(with_counts=False)


def _dense_in_body(x_ref, wl_ref, wr_ref, b_ref, p_ref, r_ref):
    xv = x_ref[...]
    p_ref[...] = lax.dot(xv, wl_ref[...], precision=lax.Precision.HIGHEST,
                         preferred_element_type=_f32)
    r_ref[...] = lax.dot(xv, wr_ref[...], precision=lax.Precision.HIGHEST,
                         preferred_element_type=_f32) + b_ref[...]


_dense_in = pl.pallas_call(
    _dense_in_body,
    out_shape=(jax.ShapeDtypeStruct((NN, HID), _f32),
               jax.ShapeDtypeStruct((NN, HID), _f32)),
)


def _mid_body(s_ref, c_ref, r_ref, wl_ref, wr_ref, b_ref, p_ref, r2_ref):
    s = s_ref[0, :NN, :] + s_ref[1, :NN, :]
    cnt = c_ref[0, :NN, 0:1] + c_ref[1, :NN, 0:1]
    mean = s / jnp.maximum(cnt, 1.0)
    h = jnp.maximum(mean + r_ref[...], 0.0)
    p_ref[...] = lax.dot(h, wl_ref[...], precision=lax.Precision.HIGHEST,
                         preferred_element_type=_f32)
    r2_ref[...] = lax.dot(h, wr_ref[...], precision=lax.Precision.HIGHEST,
                          preferred_element_type=_f32) + b_ref[...]


_mid = pl.pallas_call(
    _mid_body,
    out_shape=(jax.ShapeDtypeStruct((NN, HID), _f32),
               jax.ShapeDtypeStruct((NN, HID), _f32)),
)


def _head_body(s_ref, c_ref, r_ref, w_ref, b_ref, o_ref):
    s = s_ref[0, :NN, :] + s_ref[1, :NN, :]
    cnt = c_ref[0, :NN, 0:1] + c_ref[1, :NN, 0:1]
    mean = s / jnp.maximum(cnt, 1.0)
    h = jnp.maximum(mean + r_ref[...], 0.0)
    o_ref[...] = lax.dot(h, w_ref[...], precision=lax.Precision.HIGHEST,
                         preferred_element_type=_f32) + b_ref[...]


_head = pl.pallas_call(
    _head_body,
    out_shape=jax.ShapeDtypeStruct((NN, 11), _f32),
)


def kernel(x, edge_index, W1l, b1, W1r, W2l, b2, W2r, Wh, bh, Wm, bm):
    src = edge_index[0]
    dst = edge_index[1]
    npad = EPAD - EDGES
    pad_iota = jnp.arange(npad, dtype=jnp.int32)
    src_r = jnp.concatenate(
        [src, pad_iota % NN]).reshape(NW, CPW, CHUNK)
    dst_r = jnp.concatenate(
        [dst, NN + pad_iota % (NPAD - NN)]).reshape(NW, CPW, CHUNK)

    p1, r1 = _dense_in(x, W1l, W1r, b1.reshape(1, HID))
    s1, cnt = _edge_pass1(p1, src_r, dst_r)
    p2, r2 = _mid(s1, cnt, r1, W2l, W2r, b2.reshape(1, HID))
    s2 = _edge_pass2(p2, src_r, dst_r)
    whm = jnp.concatenate([Wh, Wm], axis=1)
    bhm = jnp.concatenate([bh, bm]).reshape(1, 11)
    o = _head(s2, cnt, r2, whm, bhm)
    return (o[:, :3], o[:, 3:11])


# speedup vs baseline: 21.4580x; 1.1719x over previous
import jax
import jax.numpy as jnp
from jax import lax
from jax.experimental import pallas as pl
from jax.experimental.pallas import tpu as pltpu
from jax.experimental.pallas import tpu_sc as plsc

NN = 10000
EDGES = 320000
DIN = 128
HID = 32
NC, NS = 2, 16
NW = NC * NS
CHUNK = 128
CPW = 80
NBUF = 4
EPAD = NW * CPW * CHUNK
NPAD = 10240
RPW = NPAD // NS
PK = 128 // HID
PR = NPAD // PK

_f32 = jnp.float32


def _edge_pass(with_counts):
    out_types = [jax.ShapeDtypeStruct((NC, NPAD, HID), _f32)]
    scratch = [
        pltpu.VMEM((CPW, CHUNK), jnp.int32),
        pltpu.VMEM((CPW, CHUNK), jnp.int32),
    ]
    scratch += [pltpu.VMEM((CHUNK, HID), _f32) for _ in range(NBUF)]
    scratch.append(pltpu.VMEM((RPW, HID), _f32))
    scratch += [pltpu.SemaphoreType.DMA] * NBUF
    if with_counts:
        out_types.append(jax.ShapeDtypeStruct((NC, NPAD, HID), _f32))
        scratch.append(pltpu.VMEM((CHUNK, HID), _f32))
    scratch.append(pltpu.VMEM_SHARED((NPAD, HID), _f32))
    if with_counts:
        scratch.append(pltpu.VMEM_SHARED((NPAD, HID), _f32))

    def body(*refs):
        if with_counts:
            (table, srci, dsti, out, cnt_out,
             src_v, dst_v, *mid, ones_v, acc, cntacc) = refs
        else:
            (table, srci, dsti, out,
             src_v, dst_v, *mid, acc) = refs
        rows = mid[:NBUF]
        zbuf = mid[NBUF]
        gsem = mid[NBUF + 1:NBUF + 1 + NBUF]

        cid = lax.axis_index("c")
        sid = lax.axis_index("s")
        wid = cid * NS + sid

        @pl.loop(0, RPW)
        def _(i):
            zbuf.at[i, pl.ds(0, 16)][...] = jnp.zeros((16,), _f32)
            zbuf.at[i, pl.ds(16, 16)][...] = jnp.zeros((16,), _f32)

        pltpu.sync_copy(zbuf, acc.at[pl.ds(sid * RPW, RPW)])
        if with_counts:
            pltpu.sync_copy(zbuf, cntacc.at[pl.ds(sid * RPW, RPW)])

            @pl.loop(0, CHUNK)
            def _(i):
                ones_v.at[i, pl.ds(0, 16)][...] = jnp.ones((16,), _f32)
                ones_v.at[i, pl.ds(16, 16)][...] = jnp.ones((16,), _f32)

        pltpu.sync_copy(srci.at[wid], src_v)
        pltpu.sync_copy(dsti.at[wid], dst_v)

        plsc.subcore_barrier()

        for b in range(NBUF):
            pltpu.async_copy(table.at[src_v.at[b]], rows[b], gsem[b])

        @pl.loop(0, CPW, step=NBUF)
        def _(j):
            for b in range(NBUF):
                c = j + b
                pltpu.make_async_copy(
                    table.at[src_v.at[c]], rows[b], gsem[b]).wait()
                pltpu.sync_copy(rows[b], acc.at[dst_v.at[c]], add=True)

                @pl.when(c + NBUF < CPW)
                def _():
                    pltpu.async_copy(
                        table.at[src_v.at[c + NBUF]], rows[b], gsem[b])

                if with_counts:
                    pltpu.sync_copy(ones_v, cntacc.at[dst_v.at[c]], add=True)

        plsc.subcore_barrier()

        pltpu.sync_copy(acc.at[pl.ds(sid * RPW, RPW)],
                        out.at[cid, pl.ds(sid * RPW, RPW)])
        if with_counts:
            pltpu.sync_copy(cntacc.at[pl.ds(sid * RPW, RPW)],
                            cnt_out.at[cid, pl.ds(sid * RPW, RPW)])

    mesh = plsc.VectorSubcoreMesh(
        core_axis_name="c", subcore_axis_name="s", num_cores=NC, num_subcores=NS
    )
    return pl.kernel(
        body,
        out_type=tuple(out_types) if with_counts else out_types[0],
        mesh=mesh,
        scratch_types=scratch,
        compiler_params=pltpu.CompilerParams(use_tc_tiling_on_sc=False),
    )


_edge_pass1 = _edge_pass(with_counts=True)
_edge_pass2 = _edge_pass(with_counts=False)


def _pack(a):
    af = jnp.concatenate([a, jnp.zeros((NPAD - NN, HID), _f32)])
    af3 = af.reshape(PR, PK, HID)
    return jnp.concatenate([af3[:, k, :] for k in range(PK)], axis=1)


def _dense_in_body(x_ref, wl_ref, wr_ref, b_ref, p_ref, r_ref):
    xv = x_ref[...]
    p = lax.dot(xv, wl_ref[...], precision=lax.Precision.HIGHEST,
                preferred_element_type=_f32)
    r = lax.dot(xv, wr_ref[...], precision=lax.Precision.HIGHEST,
                preferred_element_type=_f32) + b_ref[...]
    p_ref[...] = _pack(p)
    r_ref[...] = _pack(r)


_dense_in = pl.pallas_call(
    _dense_in_body,
    out_shape=(jax.ShapeDtypeStruct((PR, 128), _f32),
               jax.ShapeDtypeStruct((PR, 128), _f32)),
)


def _mid_body(s_ref, c_ref, r_ref, wl_ref, wr_ref, b_ref, p_ref, r2_ref):
    s = s_ref[0] + s_ref[1]
    cnt = c_ref[0] + c_ref[1]
    h = jnp.maximum(s / jnp.maximum(cnt, 1.0) + r_ref[...], 0.0)
    p_ref[...] = lax.dot(h, wl_ref[...], precision=lax.Precision.HIGHEST,
                         preferred_element_type=_f32)
    r2_ref[...] = lax.dot(h, wr_ref[...], precision=lax.Precision.HIGHEST,
                          preferred_element_type=_f32) + b_ref[...]


_mid = pl.pallas_call(
    _mid_body,
    out_shape=(jax.ShapeDtypeStruct((PR, 128), _f32),
               jax.ShapeDtypeStruct((PR, 128), _f32)),
)


def _head_body(s_ref, c_ref, r_ref, wh_ref, bh_ref, wm_ref, bm_ref,
               o3_ref, o8_ref):
    s = s_ref[0] + s_ref[1]
    cnt = c_ref[0] + c_ref[1]
    h_pk = jnp.maximum(s / jnp.maximum(cnt, 1.0) + r_ref[...], 0.0)
    h4 = jnp.stack([h_pk[:, k * HID:(k + 1) * HID] for k in range(PK)],
                   axis=1)
    h = h4.reshape(NPAD, HID)[:NN]
    o3_ref[...] = lax.dot(h, wh_ref[...], precision=lax.Precision.HIGHEST,
                          preferred_element_type=_f32) + bh_ref[...]
    o8_ref[...] = lax.dot(h, wm_ref[...], precision=lax.Precision.HIGHEST,
                          preferred_element_type=_f32) + bm_ref[...]


_head = pl.pallas_call(
    _head_body,
    out_shape=(jax.ShapeDtypeStruct((NN, 3), _f32),
               jax.ShapeDtypeStruct((NN, 8), _f32)),
)


def kernel(x, edge_index, W1l, b1, W1r, W2l, b2, W2r, Wh, bh, Wm, bm):
    src = edge_index[0]
    dst = edge_index[1]
    npad = EPAD - EDGES
    pad_iota = jnp.arange(npad, dtype=jnp.int32)
    src_r = jnp.concatenate(
        [src, pad_iota % NN]).reshape(NW, CPW, CHUNK)
    dst_r = jnp.concatenate(
        [dst, NN + pad_iota % (NPAD - NN)]).reshape(NW, CPW, CHUNK)

    eye4 = jnp.eye(PK, dtype=_f32)
    w2l4 = jnp.kron(eye4, W2l)
    w2r4 = jnp.kron(eye4, W2r)
    b2_4 = jnp.tile(b2, PK).reshape(1, 128)

    p1, r1 = _dense_in(x, W1l, W1r, b1.reshape(1, HID))
    s1, cnt = _edge_pass1(p1.reshape(NPAD, HID), src_r, dst_r)
    s1_pk = s1.reshape(NC, PR, 128)
    cnt_pk = cnt.reshape(NC, PR, 128)
    p2, r2 = _mid(s1_pk, cnt_pk, r1, w2l4, w2r4, b2_4)
    s2 = _edge_pass2(p2.reshape(NPAD, HID), src_r, dst_r)
    return _head(s2.reshape(NC, PR, 128), cnt_pk, r2,
                 Wh, bh.reshape(1, 3), Wm, bm.reshape(1, 8))
